# Initial kernel scaffold; baseline (speedup 1.0000x reference)
import jax
import jax.numpy as jnp
from jax.experimental import pallas as pl


def kernel(batch_targets):
    raise NotImplementedError("write your pallas kernel here")


# speedup vs baseline: 3.7097x; 3.7097x over previous
import numpy as np
import jax
import jax.numpy as jnp
from jax.experimental import pallas as pl
from jax.experimental.pallas import tpu as pltpu

_B = 16
_T = 50
_AM = 3
_GH = 76
_GW = 76
_BODY = 85
_HW = _GH * _GW
_C = _AM * _HW

_SA = (np.array(
    [[10.0, 13.0], [16.0, 30.0], [33.0, 23.0], [30.0, 61.0], [62.0, 45.0],
     [59.0, 119.0], [116.0, 90.0], [156.0, 198.0], [373.0, 326.0]],
    dtype=np.float32) / np.float32(8.0))
_SA_PROD = (_SA[:, 0] * _SA[:, 1]).astype(np.float32)


def _yolo_body(bt_ref, gt_ref, no_ref):
    gt_ref[...] = jnp.zeros((1, _C, _BODY), jnp.float32)
    no_ref[...] = jnp.ones((1, _C, 1), jnp.float32)
    v85 = jax.lax.broadcasted_iota(jnp.int32, (1, _BODY), 1)

    def body(t, carry):
        x = bt_ref[0, t, 0] * np.float32(_GH)
        y = bt_ref[0, t, 1] * np.float32(_GW)
        w = bt_ref[0, t, 2] * np.float32(_GH)
        h = bt_ref[0, t, 3] * np.float32(_GW)
        c = bt_ref[0, t, 4]
        wh = w * h
        best = jnp.int32(0)
        best_iou = jnp.float32(-1.0)
        for a in range(9):
            inter = jnp.minimum(w, _SA[a, 0]) * jnp.minimum(h, _SA[a, 1])
            union = wh + _SA_PROD[a] - inter
            iou = inter / union
            take = iou > best_iou
            best = jnp.where(take, jnp.int32(a), best)
            best_iou = jnp.where(take, iou, best_iou)
        valid = best < _AM
        i = jnp.floor(x).astype(jnp.int32)
        j = jnp.floor(y).astype(jnp.int32)
        cls = c.astype(jnp.int32) + 5
        cell = best * _HW + j * _GW + i
        row = jnp.where(
            v85 == cls, 1.0,
            jnp.where(v85 == 4, 1.0,
                      jnp.where(v85 == 3, h,
                                jnp.where(v85 == 2, w,
                                          jnp.where(v85 == 1, y,
                                                    jnp.where(v85 == 0, x,
                                                              0.0))))))
        row = row.astype(jnp.float32)

        @pl.when(valid)
        def _():
            gt_ref[0, pl.ds(cell, 1), :] = row
            no_ref[0, pl.ds(cell, 1), :] = jnp.zeros((1, 1), jnp.float32)

        return carry

    jax.lax.fori_loop(0, _T, body, jnp.int32(0))


def kernel(batch_targets):
    gt_flat, no_flat = pl.pallas_call(
        _yolo_body,
        grid=(_B,),
        in_specs=[
            pl.BlockSpec((1, _T, 5), lambda b: (b, 0, 0),
                         memory_space=pltpu.SMEM),
        ],
        out_specs=[
            pl.BlockSpec((1, _C, _BODY), lambda b: (b, 0, 0)),
            pl.BlockSpec((1, _C, 1), lambda b: (b, 0, 0)),
        ],
        out_shape=[
            jax.ShapeDtypeStruct((_B, _C, _BODY), jnp.float32),
            jax.ShapeDtypeStruct((_B, _C, 1), jnp.float32),
        ],
    )(batch_targets)
    gt = gt_flat.reshape(_B, _AM, _GH, _GW, _BODY)
    no_obj = no_flat.reshape(_B, _AM, _GH, _GW)
    return gt, no_obj


# speedup vs baseline: 5.7368x; 1.5464x over previous
import numpy as np
import jax
import jax.numpy as jnp
from jax.experimental import pallas as pl
from jax.experimental.pallas import tpu as pltpu

_B = 16
_T = 50
_AM = 3
_GH = 76
_GW = 76
_BODY = 85

_SA = (np.array(
    [[10.0, 13.0], [16.0, 30.0], [33.0, 23.0], [30.0, 61.0], [62.0, 45.0],
     [59.0, 119.0], [116.0, 90.0], [156.0, 198.0], [373.0, 326.0]],
    dtype=np.float32) / np.float32(8.0))
_SA_PROD = (_SA[:, 0] * _SA[:, 1]).astype(np.float32)


def _yolo_body(bt_ref, gt_ref, no_ref):
    gt_ref[...] = jnp.zeros((1, _AM, _GH, _GW, _BODY), jnp.float32)
    no_ref[...] = jnp.ones((1, _AM, _GH, _GW), jnp.float32)
    v85 = jax.lax.broadcasted_iota(jnp.int32, (1, _BODY), 1)
    v76 = jax.lax.broadcasted_iota(jnp.int32, (1, _GW), 1)

    def body(t, carry):
        x = bt_ref[0, t, 0] * np.float32(_GH)
        y = bt_ref[0, t, 1] * np.float32(_GW)
        w = bt_ref[0, t, 2] * np.float32(_GH)
        h = bt_ref[0, t, 3] * np.float32(_GW)
        c = bt_ref[0, t, 4]
        wh = w * h
        best = jnp.int32(0)
        best_iou = jnp.float32(-1.0)
        for a in range(9):
            inter = jnp.minimum(w, _SA[a, 0]) * jnp.minimum(h, _SA[a, 1])
            union = wh + _SA_PROD[a] - inter
            iou = inter / union
            take = iou > best_iou
            best = jnp.where(take, jnp.int32(a), best)
            best_iou = jnp.where(take, iou, best_iou)
        valid = best < _AM
        i = jnp.floor(x).astype(jnp.int32)
        j = jnp.floor(y).astype(jnp.int32)
        cls = c.astype(jnp.int32) + 5
        row = jnp.where(
            v85 == cls, 1.0,
            jnp.where(v85 == 4, 1.0,
                      jnp.where(v85 == 3, h,
                                jnp.where(v85 == 2, w,
                                          jnp.where(v85 == 1, y,
                                                    jnp.where(v85 == 0, x,
                                                              0.0))))))
        row = row.astype(jnp.float32)

        @pl.when(valid)
        def _():
            gt_ref[0, best, j, pl.ds(i, 1), :] = row
            no_row = no_ref[0, best, pl.ds(j, 1), :]
            no_ref[0, best, pl.ds(j, 1), :] = jnp.where(v76 == i, 0.0, no_row)

        return carry

    jax.lax.fori_loop(0, _T, body, jnp.int32(0))


def kernel(batch_targets):
    gt, no_obj = pl.pallas_call(
        _yolo_body,
        grid=(_B,),
        in_specs=[
            pl.BlockSpec((1, _T, 5), lambda b: (b, 0, 0),
                         memory_space=pltpu.SMEM),
        ],
        out_specs=[
            pl.BlockSpec((1, _AM, _GH, _GW, _BODY), lambda b: (b, 0, 0, 0, 0)),
            pl.BlockSpec((1, _AM, _GH, _GW), lambda b: (b, 0, 0, 0)),
        ],
        out_shape=[
            jax.ShapeDtypeStruct((_B, _AM, _GH, _GW, _BODY), jnp.float32),
            jax.ShapeDtypeStruct((_B, _AM, _GH, _GW), jnp.float32),
        ],
    )(batch_targets)
    return gt, no_obj


# speedup vs baseline: 6.5858x; 1.1480x over previous
import numpy as np
import jax
import jax.numpy as jnp
from jax.experimental import pallas as pl
from jax.experimental.pallas import tpu as pltpu

_B = 16
_T = 50
_AM = 3
_GH = 76
_GW = 76
_BODY = 85
_N = _B * _T

_SA = (np.array(
    [[10.0, 13.0], [16.0, 30.0], [33.0, 23.0], [30.0, 61.0], [62.0, 45.0],
     [59.0, 119.0], [116.0, 90.0], [156.0, 198.0], [373.0, 326.0]],
    dtype=np.float32) / np.float32(8.0))
_SA_PROD = (_SA[:, 0] * _SA[:, 1]).astype(np.float32)


def _prepass_body(bt_ref, rows_ref, sc_ref):
    t5 = bt_ref[...]
    x = t5[:, 0:1] * np.float32(_GH)
    y = t5[:, 1:2] * np.float32(_GW)
    w = t5[:, 2:3] * np.float32(_GH)
    h = t5[:, 3:4] * np.float32(_GW)
    c = t5[:, 4:5]
    wh = w * h
    best = jnp.zeros((_N, 1), jnp.int32)
    best_iou = jnp.full((_N, 1), -1.0, jnp.float32)
    for a in range(9):
        inter = jnp.minimum(w, _SA[a, 0]) * jnp.minimum(h, _SA[a, 1])
        union = wh + _SA_PROD[a] - inter
        iou = inter / union
        take = iou > best_iou
        best = jnp.where(take, jnp.int32(a), best)
        best_iou = jnp.where(take, iou, best_iou)
    valid = best < _AM
    i = jnp.floor(x)
    j = jnp.floor(y)
    cls = c.astype(jnp.int32) + 5
    v85 = jax.lax.broadcasted_iota(jnp.int32, (_N, _BODY), 1)
    row = jnp.where(
        v85 == cls, 1.0,
        jnp.where(v85 == 4, 1.0,
                  jnp.where(v85 == 3, h,
                            jnp.where(v85 == 2, w,
                                      jnp.where(v85 == 1, y,
                                                jnp.where(v85 == 0, x,
                                                          0.0))))))
    rows_ref[...] = row.astype(jnp.float32)
    sc_ref[...] = jnp.concatenate(
        [best.astype(jnp.float32), j, i,
         valid.astype(jnp.float32)], axis=1)


def _main_body(sc_ref, rows_ref, gt_ref, no_ref):
    gt_ref[...] = jnp.zeros((1, _AM, _GH, _GW, _BODY), jnp.float32)
    no_ref[...] = jnp.ones((1, _AM, _GH, _GW), jnp.float32)
    v76 = jax.lax.broadcasted_iota(jnp.int32, (1, _GW), 1)

    def body(t, carry):
        valid = sc_ref[0, t, 3] > 0.5
        k = sc_ref[0, t, 0].astype(jnp.int32)
        j = sc_ref[0, t, 1].astype(jnp.int32)
        i = sc_ref[0, t, 2].astype(jnp.int32)

        @pl.when(valid)
        def _():
            gt_ref[0, k, j, pl.ds(i, 1), :] = rows_ref[0, pl.ds(t, 1), :]
            no_row = no_ref[0, k, pl.ds(j, 1), :]
            no_ref[0, k, pl.ds(j, 1), :] = jnp.where(v76 == i, 0.0, no_row)

        return carry

    jax.lax.fori_loop(0, _T, body, jnp.int32(0))


def kernel(batch_targets):
    bt_flat = batch_targets.reshape(_N, 5)
    rows, sc = pl.pallas_call(
        _prepass_body,
        out_shape=[
            jax.ShapeDtypeStruct((_N, _BODY), jnp.float32),
            jax.ShapeDtypeStruct((_N, 4), jnp.float32),
        ],
    )(bt_flat)
    rows = rows.reshape(_B, _T, _BODY)
    sc = sc.reshape(_B, _T, 4)
    gt, no_obj = pl.pallas_call(
        _main_body,
        grid=(_B,),
        in_specs=[
            pl.BlockSpec((1, _T, 4), lambda b: (b, 0, 0),
                         memory_space=pltpu.SMEM),
            pl.BlockSpec((1, _T, _BODY), lambda b: (b, 0, 0)),
        ],
        out_specs=[
            pl.BlockSpec((1, _AM, _GH, _GW, _BODY), lambda b: (b, 0, 0, 0, 0)),
            pl.BlockSpec((1, _AM, _GH, _GW), lambda b: (b, 0, 0, 0)),
        ],
        out_shape=[
            jax.ShapeDtypeStruct((_B, _AM, _GH, _GW, _BODY), jnp.float32),
            jax.ShapeDtypeStruct((_B, _AM, _GH, _GW), jnp.float32),
        ],
    )(sc, rows)
    return gt, no_obj


# speedup vs baseline: 6.7218x; 1.0206x over previous
import numpy as np
import jax
import jax.numpy as jnp
from jax.experimental import pallas as pl
from jax.experimental.pallas import tpu as pltpu

_B = 16
_T = 50
_TP = 64
_AM = 3
_GH = 76
_GW = 76
_BODY = 85
_N = _B * _TP

_SA = (np.array(
    [[10.0, 13.0], [16.0, 30.0], [33.0, 23.0], [30.0, 61.0], [62.0, 45.0],
     [59.0, 119.0], [116.0, 90.0], [156.0, 198.0], [373.0, 326.0]],
    dtype=np.float32) / np.float32(8.0))
_SA_PROD = (_SA[:, 0] * _SA[:, 1]).astype(np.float32)


def _prepass_body(bt_ref, rows_ref, sc_ref):
    t5 = bt_ref[...]
    x = t5[:, 0:1] * np.float32(_GH)
    y = t5[:, 1:2] * np.float32(_GW)
    w = t5[:, 2:3] * np.float32(_GH)
    h = t5[:, 3:4] * np.float32(_GW)
    c = t5[:, 4:5]
    wh = w * h
    best = jnp.zeros((_N, 1), jnp.int32)
    best_iou = jnp.full((_N, 1), -1.0, jnp.float32)
    for a in range(9):
        inter = jnp.minimum(w, _SA[a, 0]) * jnp.minimum(h, _SA[a, 1])
        union = wh + _SA_PROD[a] - inter
        iou = inter / union
        take = iou > best_iou
        best = jnp.where(take, jnp.int32(a), best)
        best_iou = jnp.where(take, iou, best_iou)
    slot = jax.lax.broadcasted_iota(jnp.int32, (_N, 1), 0)
    valid = (best < _AM) & (slot % _TP < _T)
    i = jnp.floor(x)
    j = jnp.floor(y)
    cls = c.astype(jnp.int32) + 5
    v85 = jax.lax.broadcasted_iota(jnp.int32, (_N, _BODY), 1)
    row = jnp.where(
        v85 == cls, 1.0,
        jnp.where(v85 == 4, 1.0,
                  jnp.where(v85 == 3, h,
                            jnp.where(v85 == 2, w,
                                      jnp.where(v85 == 1, y,
                                                jnp.where(v85 == 0, x,
                                                          0.0))))))
    rows_ref[...] = row.astype(jnp.float32)
    sc_ref[...] = jnp.concatenate(
        [best.astype(jnp.float32), j, i,
         valid.astype(jnp.float32)], axis=1)


def _main_body(sc_ref, rows_ref, gt_ref, no_ref):
    gt_ref[...] = jnp.zeros((1, _AM, _GH, _GW, _BODY), jnp.float32)
    no_ref[...] = jnp.ones((1, _AM, _GH, _GW), jnp.float32)
    v76 = jax.lax.broadcasted_iota(jnp.int32, (1, _GW), 1)

    def body(t, carry):
        valid = sc_ref[t, 3] > 0.5
        k = sc_ref[t, 0].astype(jnp.int32)
        j = sc_ref[t, 1].astype(jnp.int32)
        i = sc_ref[t, 2].astype(jnp.int32)

        @pl.when(valid)
        def _():
            gt_ref[0, k, j, pl.ds(i, 1), :] = rows_ref[pl.ds(t, 1), :]
            no_row = no_ref[0, k, pl.ds(j, 1), :]
            no_ref[0, k, pl.ds(j, 1), :] = jnp.where(v76 == i, 0.0, no_row)

        return carry

    jax.lax.fori_loop(0, _T, body, jnp.int32(0))


def kernel(batch_targets):
    bt_pad = jnp.pad(batch_targets, ((0, 0), (0, _TP - _T), (0, 0)))
    bt_flat = bt_pad.reshape(_N, 5)
    rows, sc = pl.pallas_call(
        _prepass_body,
        out_shape=[
            jax.ShapeDtypeStruct((_N, _BODY), jnp.float32),
            jax.ShapeDtypeStruct((_N, 4), jnp.float32),
        ],
    )(bt_flat)
    gt, no_obj = pl.pallas_call(
        _main_body,
        grid=(_B,),
        in_specs=[
            pl.BlockSpec((_TP, 4), lambda b: (b, 0),
                         memory_space=pltpu.SMEM),
            pl.BlockSpec((_TP, _BODY), lambda b: (b, 0)),
        ],
        out_specs=[
            pl.BlockSpec((1, _AM, _GH, _GW, _BODY), lambda b: (b, 0, 0, 0, 0)),
            pl.BlockSpec((1, _AM, _GH, _GW), lambda b: (b, 0, 0, 0)),
        ],
        out_shape=[
            jax.ShapeDtypeStruct((_B, _AM, _GH, _GW, _BODY), jnp.float32),
            jax.ShapeDtypeStruct((_B, _AM, _GH, _GW), jnp.float32),
        ],
    )(sc, rows)
    return gt, no_obj


# speedup vs baseline: 6.7393x; 1.0026x over previous
import numpy as np
import jax
import jax.numpy as jnp
from jax.experimental import pallas as pl
from jax.experimental.pallas import tpu as pltpu

_B = 16
_T = 50
_AM = 3
_GH = 76
_GW = 76
_BODY = 85

_SA = (np.array(
    [[10.0, 13.0], [16.0, 30.0], [33.0, 23.0], [30.0, 61.0], [62.0, 45.0],
     [59.0, 119.0], [116.0, 90.0], [156.0, 198.0], [373.0, 326.0]],
    dtype=np.float32) / np.float32(8.0))
_SA_PROD = (_SA[:, 0] * _SA[:, 1]).astype(np.float32)


def _prepass_body(bt_ref, rows_ref, sc_ref):
    t5 = bt_ref[...]
    x = t5[:, :, 0:1] * np.float32(_GH)
    y = t5[:, :, 1:2] * np.float32(_GW)
    w = t5[:, :, 2:3] * np.float32(_GH)
    h = t5[:, :, 3:4] * np.float32(_GW)
    c = t5[:, :, 4:5]
    wh = w * h
    best = jnp.zeros((_B, _T, 1), jnp.int32)
    best_iou = jnp.full((_B, _T, 1), -1.0, jnp.float32)
    for a in range(9):
        inter = jnp.minimum(w, _SA[a, 0]) * jnp.minimum(h, _SA[a, 1])
        union = wh + _SA_PROD[a] - inter
        iou = inter / union
        take = iou > best_iou
        best = jnp.where(take, jnp.int32(a), best)
        best_iou = jnp.where(take, iou, best_iou)
    valid = best < _AM
    i = jnp.floor(x)
    j = jnp.floor(y)
    cls = c.astype(jnp.int32) + 5
    v85 = jax.lax.broadcasted_iota(jnp.int32, (_B, _T, _BODY), 2)
    row = jnp.where(
        v85 == cls, 1.0,
        jnp.where(v85 == 4, 1.0,
                  jnp.where(v85 == 3, h,
                            jnp.where(v85 == 2, w,
                                      jnp.where(v85 == 1, y,
                                                jnp.where(v85 == 0, x,
                                                          0.0))))))
    rows_ref[...] = row.astype(jnp.float32)
    sc_ref[...] = jnp.concatenate(
        [best.astype(jnp.float32), j, i,
         valid.astype(jnp.float32)], axis=2)


def _main_body(sc_ref, rows_ref, gt_ref, no_ref):
    gt_ref[...] = jnp.zeros((1, _AM, _GH, _GW, _BODY), jnp.float32)
    no_ref[...] = jnp.ones((1, _AM, _GH, _GW), jnp.float32)
    v76 = jax.lax.broadcasted_iota(jnp.int32, (1, _GW), 1)

    def body(t, carry):
        valid = sc_ref[0, t, 3] > 0.5
        k = sc_ref[0, t, 0].astype(jnp.int32)
        j = sc_ref[0, t, 1].astype(jnp.int32)
        i = sc_ref[0, t, 2].astype(jnp.int32)

        @pl.when(valid)
        def _():
            gt_ref[0, k, j, pl.ds(i, 1), :] = rows_ref[0, pl.ds(t, 1), :]
            no_row = no_ref[0, k, pl.ds(j, 1), :]
            no_ref[0, k, pl.ds(j, 1), :] = jnp.where(v76 == i, 0.0, no_row)

        return carry

    jax.lax.fori_loop(0, _T, body, jnp.int32(0))


def kernel(batch_targets):
    rows, sc = pl.pallas_call(
        _prepass_body,
        out_shape=[
            jax.ShapeDtypeStruct((_B, _T, _BODY), jnp.float32),
            jax.ShapeDtypeStruct((_B, _T, 4), jnp.float32),
        ],
    )(batch_targets)
    gt, no_obj = pl.pallas_call(
        _main_body,
        grid=(_B,),
        in_specs=[
            pl.BlockSpec((1, _T, 4), lambda b: (b, 0, 0),
                         memory_space=pltpu.SMEM),
            pl.BlockSpec((1, _T, _BODY), lambda b: (b, 0, 0)),
        ],
        out_specs=[
            pl.BlockSpec((1, _AM, _GH, _GW, _BODY), lambda b: (b, 0, 0, 0, 0)),
            pl.BlockSpec((1, _AM, _GH, _GW), lambda b: (b, 0, 0, 0)),
        ],
        out_shape=[
            jax.ShapeDtypeStruct((_B, _AM, _GH, _GW, _BODY), jnp.float32),
            jax.ShapeDtypeStruct((_B, _AM, _GH, _GW), jnp.float32),
        ],
    )(sc, rows)
    return gt, no_obj


# speedup vs baseline: 6.8143x; 1.0111x over previous
import numpy as np
import jax
import jax.numpy as jnp
from jax.experimental import pallas as pl
from jax.experimental.pallas import tpu as pltpu

_B = 16
_T = 50
_AM = 3
_GH = 76
_GW = 76
_BODY = 85

_SA = (np.array(
    [[10.0, 13.0], [16.0, 30.0], [33.0, 23.0], [30.0, 61.0], [62.0, 45.0],
     [59.0, 119.0], [116.0, 90.0], [156.0, 198.0], [373.0, 326.0]],
    dtype=np.float32) / np.float32(8.0))
_SA_PROD = (_SA[:, 0] * _SA[:, 1]).astype(np.float32)


def _prepass_body(bt_ref, rows_ref, sc_ref):
    t5 = bt_ref[...]
    x = t5[:, :, 0:1] * np.float32(_GH)
    y = t5[:, :, 1:2] * np.float32(_GW)
    w = t5[:, :, 2:3] * np.float32(_GH)
    h = t5[:, :, 3:4] * np.float32(_GW)
    c = t5[:, :, 4:5]
    wh = w * h
    best = jnp.zeros((_B, _T, 1), jnp.int32)
    best_iou = jnp.full((_B, _T, 1), -1.0, jnp.float32)
    for a in range(9):
        inter = jnp.minimum(w, _SA[a, 0]) * jnp.minimum(h, _SA[a, 1])
        union = wh + _SA_PROD[a] - inter
        iou = inter / union
        take = iou > best_iou
        best = jnp.where(take, jnp.int32(a), best)
        best_iou = jnp.where(take, iou, best_iou)
    valid = best < _AM
    i = jnp.floor(x)
    j = jnp.floor(y)
    cls = c.astype(jnp.int32) + 5
    v85 = jax.lax.broadcasted_iota(jnp.int32, (_B, _T, _BODY), 2)
    row = jnp.where(
        v85 == cls, 1.0,
        jnp.where(v85 == 4, 1.0,
                  jnp.where(v85 == 3, h,
                            jnp.where(v85 == 2, w,
                                      jnp.where(v85 == 1, y,
                                                jnp.where(v85 == 0, x,
                                                          0.0))))))
    rows_ref[...] = row.astype(jnp.float32)
    sc_ref[...] = jnp.concatenate(
        [best.astype(jnp.float32), j, i,
         valid.astype(jnp.float32)], axis=2)


def _main_body(sc_ref, scp_ref, rows_ref, gt_hbm, no_hbm,
               slab_gt, slab_no, sem_gt, sem_no):
    p = pl.program_id(0)
    s = jax.lax.rem(p, 2)
    v76 = jax.lax.broadcasted_iota(jnp.int32, (1, _GW), 1)

    @pl.when(p < 2)
    def _():
        slab_gt[pl.ds(s, 1)] = jnp.zeros((1, _AM, _GH, _GW, _BODY),
                                         jnp.float32)
        slab_no[pl.ds(s, 1)] = jnp.ones((1, _AM, _GH, _GW), jnp.float32)

    @pl.when(p >= 2)
    def _():
        pltpu.make_async_copy(slab_gt.at[s], gt_hbm.at[p - 2],
                              sem_gt.at[s]).wait()
        pltpu.make_async_copy(slab_no.at[s], no_hbm.at[p - 2],
                              sem_no.at[s]).wait()

        def restore(t, carry):
            valid = scp_ref[0, t, 3] > 0.5
            k = scp_ref[0, t, 0].astype(jnp.int32)
            j = scp_ref[0, t, 1].astype(jnp.int32)
            i = scp_ref[0, t, 2].astype(jnp.int32)

            @pl.when(valid)
            def _():
                slab_gt[s, k, j, pl.ds(i, 1), :] = jnp.zeros((1, _BODY),
                                                             jnp.float32)
                no_row = slab_no[s, k, pl.ds(j, 1), :]
                slab_no[s, k, pl.ds(j, 1), :] = jnp.where(
                    v76 == i, 1.0, no_row)

            return carry

        jax.lax.fori_loop(0, _T, restore, jnp.int32(0))

    def scatter(t, carry):
        valid = sc_ref[0, t, 3] > 0.5
        k = sc_ref[0, t, 0].astype(jnp.int32)
        j = sc_ref[0, t, 1].astype(jnp.int32)
        i = sc_ref[0, t, 2].astype(jnp.int32)

        @pl.when(valid)
        def _():
            slab_gt[s, k, j, pl.ds(i, 1), :] = rows_ref[0, pl.ds(t, 1), :]
            no_row = slab_no[s, k, pl.ds(j, 1), :]
            slab_no[s, k, pl.ds(j, 1), :] = jnp.where(v76 == i, 0.0, no_row)

        return carry

    jax.lax.fori_loop(0, _T, scatter, jnp.int32(0))

    pltpu.make_async_copy(slab_gt.at[s], gt_hbm.at[p], sem_gt.at[s]).start()
    pltpu.make_async_copy(slab_no.at[s], no_hbm.at[p], sem_no.at[s]).start()

    @pl.when(p == _B - 1)
    def _():
        pltpu.make_async_copy(slab_gt.at[s], gt_hbm.at[p], sem_gt.at[s]).wait()
        pltpu.make_async_copy(slab_no.at[s], no_hbm.at[p], sem_no.at[s]).wait()
        so = 1 - s
        pltpu.make_async_copy(slab_gt.at[so], gt_hbm.at[p - 1],
                              sem_gt.at[so]).wait()
        pltpu.make_async_copy(slab_no.at[so], no_hbm.at[p - 1],
                              sem_no.at[so]).wait()


def kernel(batch_targets):
    rows, sc = pl.pallas_call(
        _prepass_body,
        out_shape=[
            jax.ShapeDtypeStruct((_B, _T, _BODY), jnp.float32),
            jax.ShapeDtypeStruct((_B, _T, 4), jnp.float32),
        ],
    )(batch_targets)
    gt, no_obj = pl.pallas_call(
        _main_body,
        grid=(_B,),
        in_specs=[
            pl.BlockSpec((1, _T, 4), lambda b: (b, 0, 0),
                         memory_space=pltpu.SMEM),
            pl.BlockSpec((1, _T, 4),
                         lambda b: (jnp.maximum(b - 2, 0), 0, 0),
                         memory_space=pltpu.SMEM),
            pl.BlockSpec((1, _T, _BODY), lambda b: (b, 0, 0)),
        ],
        out_specs=[
            pl.BlockSpec(memory_space=pl.ANY),
            pl.BlockSpec(memory_space=pl.ANY),
        ],
        out_shape=[
            jax.ShapeDtypeStruct((_B, _AM, _GH, _GW, _BODY), jnp.float32),
            jax.ShapeDtypeStruct((_B, _AM, _GH, _GW), jnp.float32),
        ],
        scratch_shapes=[
            pltpu.VMEM((2, _AM, _GH, _GW, _BODY), jnp.float32),
            pltpu.VMEM((2, _AM, _GH, _GW), jnp.float32),
            pltpu.SemaphoreType.DMA((2,)),
            pltpu.SemaphoreType.DMA((2,)),
        ],
    )(sc, sc, rows)
    return gt, no_obj
